# baseline (speedup 1.0000x reference)
import jax
import jax.numpy as jnp
from jax.experimental import pallas as pl
from jax.experimental.pallas import tpu as pltpu


def _cox_pairwise(r2, d2, e2, *, interpret=False):
    R, C = r2.shape

    def body(r_ref, d_ref, e_ref, out_ref, s_ref, er_ref):
        r = r_ref[...]
        e = e_ref[...]
        d = d_ref[...]
        gamma = jnp.max(r)
        er = jnp.exp(r - gamma)
        er_ref[...] = er
        s_ref[...] = jnp.zeros((R, C), jnp.float32)
        sub = jax.lax.broadcasted_iota(jnp.int32, (R, 1), 0)
        lane = jax.lax.broadcasted_iota(jnp.int32, (1, C), 1)

        def outer(c, _):
            ohl = (lane == c).astype(jnp.float32)
            dj = jnp.sum(d * ohl, axis=1, keepdims=True)
            erj = jnp.sum(er * ohl, axis=1, keepdims=True)
            ij = sub * C + c

            def inner(ks, _):
                dk = d_ref[pl.ds(ks, 1), :]
                ik = ks * C + lane
                m = (dj > dk) | ((dj == dk) & (ij < ik))
                part = jnp.sum(jnp.where(m, erj, 0.0), axis=0, keepdims=True)
                s_ref[pl.ds(ks, 1), :] = s_ref[pl.ds(ks, 1), :] + part
                return 0

            jax.lax.fori_loop(0, R, inner, 0, unroll=False)
            return 0

        jax.lax.fori_loop(0, C, outer, 0, unroll=False)

        S = s_ref[...] + er_ref[...]
        n_ev = jnp.sum(e)
        total = jnp.sum(e * ((r - gamma) - jnp.log(S)))
        out_ref[...] = jnp.full((1, 1), -total / jnp.maximum(n_ev, 1.0),
                                jnp.float32)

    return pl.pallas_call(
        body,
        out_shape=jax.ShapeDtypeStruct((1, 1), jnp.float32),
        scratch_shapes=[
            pltpu.VMEM((R, C), jnp.float32),
            pltpu.VMEM((R, C), jnp.float32),
        ],
        interpret=interpret,
    )(r2, d2, e2)


def kernel(risk_scores, targets):
    r = risk_scores
    if r.ndim > 1:
        r = jnp.squeeze(r, axis=1)
    d = targets[:, 0]
    e = targets[:, 1]
    B = r.shape[0]
    C = 128
    R = B // C
    out = _cox_pairwise(r.reshape(R, C), d.reshape(R, C), e.reshape(R, C))
    return out[0, 0]


# speedup vs baseline: 38.9607x; 38.9607x over previous
import functools
import jax
import jax.numpy as jnp
from jax.experimental import pallas as pl


def _body(d_ref, r_ref, e_ref, out_ref):
    D = d_ref[...]
    R = r_ref[...]
    E = e_ref[...]
    sub = jax.lax.broadcasted_iota(jnp.int32, (128, 1), 0)
    lane = jax.lax.broadcasted_iota(jnp.int32, (1, 128), 1)
    I = lane * 128 + sub
    IE = I.astype(jnp.float32) * 2.0 + E
    gamma = jnp.max(R)

    bits = [((I >> n) & 1) for n in range(14)]
    zero_bits = jnp.zeros((128, 128), jnp.int32)

    def pmat(t):
        return ((sub ^ t) == lane).astype(jnp.float32)

    def lane_partner(X, t):
        return jnp.dot(X, pmat(t), preferred_element_type=jnp.float32)

    def sub_partner(X, s):
        G = 128 // (2 * s)
        X4 = X.reshape(G, 2, s, 128)
        X4s = jnp.concatenate([X4[:, 1:2], X4[:, 0:1]], axis=1)
        return X4s.reshape(128, 128)

    for m in range(1, 15):
        bk = bits[m] if m < 14 else zero_bits
        for j_exp in range(m - 1, -1, -1):
            bj = bits[j_exp]
            keep = bk == bj
            if j_exp <= 6:
                s = 1 << j_exp
                Dq = sub_partner(D, s)
                IEq = sub_partner(IE, s)
                Rq = sub_partner(R, s)
            else:
                t = 1 << (j_exp - 7)
                Dq = lane_partner(D, t)
                IEq = lane_partner(IE, t)
                Rq = lane_partner(R, t)
            pre = (D > Dq) | ((D == Dq) & (IE < IEq))
            take = pre == keep
            D = jnp.where(take, D, Dq)
            IE = jnp.where(take, IE, IEq)
            R = jnp.where(take, R, Rq)

    er = jnp.exp(R - gamma)
    Lmat = (sub >= lane).astype(jnp.float32)
    colcum = jnp.dot(Lmat, er, preferred_element_type=jnp.float32)
    tot = colcum[127:128, :]
    Umat = (sub < lane).astype(jnp.float32)
    off = jnp.dot(tot, Umat, preferred_element_type=jnp.float32)
    S = colcum + off

    Es = (IE.astype(jnp.int32) & 1).astype(jnp.float32)
    contrib = Es * ((R - gamma) - jnp.log(S))
    n_ev = jnp.sum(Es)
    loss = -jnp.sum(contrib) / jnp.maximum(n_ev, 1.0)
    out_ref[...] = jnp.full((1, 1), loss, jnp.float32)


def _cox_sorted(d_cm, r_cm, e_cm, *, interpret=False):
    return pl.pallas_call(
        _body,
        out_shape=jax.ShapeDtypeStruct((1, 1), jnp.float32),
        interpret=interpret,
    )(d_cm, r_cm, e_cm)


def kernel(risk_scores, targets, *, interpret=False):
    r = risk_scores
    if r.ndim > 1:
        r = jnp.squeeze(r, axis=1)
    d = targets[:, 0]
    e = targets[:, 1]
    d_cm = d.reshape(128, 128).T
    r_cm = r.reshape(128, 128).T
    e_cm = e.reshape(128, 128).T
    out = _cox_sorted(d_cm, r_cm, e_cm, interpret=interpret)
    return out[0, 0]
